# baseline (speedup 1.0000x reference)
import functools

import jax
import jax.numpy as jnp
from jax import lax
from jax.experimental import pallas as pl
from jax.experimental.pallas import tpu as pltpu
from jax.experimental.pallas import tpu_sc as plsc

NC, NS, LANES = 2, 16, 16
N_NODE = 10000
NX = 2 * N_NODE
E = 320000
E_LBL = 100000
ELP = 102400
D = 128

KA = 50
EPT = E // NS
NCA = EPT // KA
MA = 5
MI = 10

KD = 80
ELPT = ELP // (NC * NS)
NCD = ELPT // KD
MD = 4

_S15 = 15 * 624

_sc_mesh = plsc.VectorSubcoreMesh(core_axis_name="c", subcore_axis_name="s")
_sc_params = pltpu.CompilerParams(needs_layout_passes=False)


@functools.partial(
    pl.kernel,
    out_type=jax.ShapeDtypeStruct((2 * 10 * NS * 1000,), jnp.float32),
    mesh=_sc_mesh,
    scratch_types=[
        pltpu.VMEM((EPT,), jnp.int32),
        pltpu.VMEM((N_NODE,), jnp.float32),
    ],
    compiler_params=_sc_params,
)
def _sc_cnt(dst_hbm, cnt_hbm, didx, hist):
    cid = lax.axis_index("c")
    sid = lax.axis_index("s")
    base = pl.multiple_of(cid * E + sid * EPT, 8)
    pltpu.sync_copy(dst_hbm.at[pl.ds(base, EPT)], didx)
    zeros16 = jnp.zeros((LANES,), jnp.float32)
    ones16 = jnp.ones((LANES,), jnp.float32)

    def _zero(i, carry):
        hist[pl.ds(i * LANES, LANES)] = zeros16
        return carry

    lax.fori_loop(0, N_NODE // LANES, _zero, 0)

    def _accum(i, carry):
        iv = didx[pl.ds(i * LANES, LANES)]
        plsc.addupdate_scatter(hist, [iv], ones16)
        return carry

    lax.fori_loop(0, EPT // LANES, _accum, 0)
    for a in range(10):
        coff = pl.multiple_of(((cid * 10 + a) * NS + sid) * 1000, 8)
        pltpu.sync_copy(hist.at[pl.ds(a * 1000, 1000)],
                        cnt_hbm.at[pl.ds(coff, 1000)])


@functools.partial(
    pl.kernel,
    out_type=jax.ShapeDtypeStruct((2, N_NODE, D), jnp.float32),
    mesh=_sc_mesh,
    scratch_types=[
        pltpu.VMEM_SHARED((N_NODE, D), jnp.float32),
        pltpu.VMEM((MI, 2, KA), jnp.int32),
        pltpu.VMEM((MA, KA, D), jnp.float32),
        pltpu.SemaphoreType.DMA((MI,)),
        pltpu.SemaphoreType.DMA((MA,)),
        pltpu.SemaphoreType.DMA((MA,)),
    ],
    compiler_params=_sc_params,
)
def _sc_agg(x_hbm, pair_hbm, zero_hbm, agg_hbm, acc, idx, rows,
            isem, gsem, ssem):
    cid = lax.axis_index("c")
    sid = lax.axis_index("s")
    off = pl.multiple_of(sid * 624, 8)

    @pl.when(sid < 15)
    def _():
        pltpu.sync_copy(zero_hbm.at[pl.ds(0, 624)], acc.at[pl.ds(off, 624)])

    @pl.when(sid == 15)
    def _():
        pltpu.sync_copy(zero_hbm, acc.at[pl.ds(_S15, 640)])

    plsc.subcore_barrier()

    def _idx_load(c, s):
        pltpu.async_copy(pair_hbm.at[cid, sid, c], idx.at[s], isem.at[s])

    def _wait_idx(c, s):
        pltpu.make_async_copy(pair_hbm.at[cid, sid, c], idx.at[s],
                              isem.at[s]).wait()

    def _gather(c, si, sr):
        pltpu.async_copy(x_hbm.at[idx.at[si, 0]], rows.at[sr], gsem.at[sr])

    def _wait_gather(c, si, sr):
        pltpu.make_async_copy(x_hbm.at[idx.at[si, 0]], rows.at[sr],
                              gsem.at[sr]).wait()

    def _scatter(c, si, sr):
        pltpu.async_copy(rows.at[sr], acc.at[idx.at[si, 1]], ssem.at[sr],
                         add=True)

    def _wait_scatter(c, si, sr):
        pltpu.make_async_copy(rows.at[sr], acc.at[idx.at[si, 1]],
                              ssem.at[sr]).wait()

    for s in range(MI - 2):
        _idx_load(s, s)
    for b in range(MA - 2):
        _wait_idx(b, b)
        _gather(b, b, b)

    def _group(g, carry):
        c0 = g * MI
        for b in range(MI):
            c = c0 + b
            sr = b % MA
            _wait_gather(c, b, sr)
            _scatter(c, b, sr)
            srg = (b + MA - 2) % MA
            sii = (b + MI - 2) % MI

            @pl.when(c >= 2)
            def _():
                _wait_scatter(c - 2, sii, srg)

            @pl.when(c + MI - 2 < NCA)
            def _():
                _idx_load(c + MI - 2, sii)

            @pl.when(c + MA - 2 < NCA)
            def _():
                _wait_idx(c + MA - 2, (b + MA - 2) % MI)
                _gather(c + MA - 2, (b + MA - 2) % MI, srg)
        return carry

    lax.fori_loop(0, NCA // MI, _group, 0)
    _wait_scatter(NCA - 2, (NCA - 2) % MI, (NCA - 2) % MA)
    _wait_scatter(NCA - 1, (NCA - 1) % MI, (NCA - 1) % MA)
    plsc.subcore_barrier()

    @pl.when(sid < 15)
    def _():
        pltpu.sync_copy(acc.at[pl.ds(off, 624)],
                        agg_hbm.at[cid, pl.ds(off, 624)])

    @pl.when(sid == 15)
    def _():
        pltpu.sync_copy(acc.at[pl.ds(_S15, 640)],
                        agg_hbm.at[cid, pl.ds(_S15, 640)])


_BM = 1000


def _tc_layer_body(x_ref, agg_ref, cnt_ref, wlt_ref, wrt_ref, bl_ref,
                   wd_ref, bd_ref, xo_ref, u_ref):
    agg = agg_ref[0]
    cnt = jnp.sum(cnt_ref[0, 0], axis=0)
    inv = 1.0 / jnp.maximum(cnt, 1.0)
    mean = agg * inv[:, None]
    dn = (((1,), (1,)), ((), ()))
    out = (lax.dot_general(mean, wlt_ref[0], dn,
                           preferred_element_type=jnp.float32)
           + bl_ref[0]
           + lax.dot_general(x_ref[...], wrt_ref[0], dn,
                             preferred_element_type=jnp.float32))
    out = jnp.maximum(out, 0.0)
    xo_ref[...] = out
    u = jnp.sum(out * wd_ref[0], axis=1) + bd_ref[pl.program_id(0)]
    u_ref[0, 0, :] = u


def _tc_layer(x, agg, cnt, wlt, wrt, bl, wd, bdv):
    bm = _BM
    nb = N_NODE // bm
    return pl.pallas_call(
        _tc_layer_body,
        grid=(2, nb),
        in_specs=[
            pl.BlockSpec((bm, D), lambda i, j: (i * (N_NODE // _BM) + j, 0)),
            pl.BlockSpec((1, bm, D), lambda i, j: (1 - i, j, 0)),
            pl.BlockSpec((1, 1, NS, 1000), lambda i, j: (1 - i, j, 0, 0)),
            pl.BlockSpec((1, D, D), lambda i, j: (1 - i, 0, 0)),
            pl.BlockSpec((1, D, D), lambda i, j: (1 - i, 0, 0)),
            pl.BlockSpec((1, 1, D), lambda i, j: (1 - i, 0, 0)),
            pl.BlockSpec((1, 1, D), lambda i, j: (i, 0, 0)),
            pl.BlockSpec(memory_space=pltpu.SMEM),
        ],
        out_specs=[
            pl.BlockSpec((bm, D), lambda i, j: (i * (N_NODE // _BM) + j, 0)),
            pl.BlockSpec((1, 1, _BM), lambda i, j: (i * (N_NODE // _BM) + j, 0, 0)),
        ],
        out_shape=[
            jax.ShapeDtypeStruct((NX, D), jnp.float32),
            jax.ShapeDtypeStruct((NX // _BM, 1, _BM), jnp.float32),
        ],
    )(x, agg, cnt, wlt, wrt, bl, wd, bdv)


_NCD_LAST = (E_LBL - 31 * ELPT) // KD


@functools.partial(
    pl.kernel,
    out_type=(
        jax.ShapeDtypeStruct((E_LBL, 2 * D), jnp.float32),
        jax.ShapeDtypeStruct((E_LBL,), jnp.float32),
    ),
    mesh=_sc_mesh,
    scratch_types=[
        pltpu.VMEM((NCD, 2, KD), jnp.int32),
        pltpu.VMEM((MD, KD, D), jnp.float32),
        pltpu.VMEM((MD, KD, D), jnp.float32),
        pltpu.VMEM((N_NODE,), jnp.float32),
        pltpu.VMEM((N_NODE,), jnp.float32),
        pltpu.VMEM((MD, KD), jnp.float32),
        pltpu.SemaphoreType.DMA((MD,)),
        pltpu.SemaphoreType.DMA((MD,)),
    ],
    compiler_params=_sc_params,
)
def _sc_dec(x_hbm, eidx_hbm, u_hbm, xcat_hbm, z_hbm,
            idx, r0, r1, u0, u1, zb, gsem, wsem):
    cid = lax.axis_index("c")
    sid = lax.axis_index("s")
    wid = sid * NC + cid
    nct = jnp.where(wid == NC * NS - 1, _NCD_LAST, NCD)
    pltpu.sync_copy(eidx_hbm.at[wid], idx)
    pltpu.sync_copy(u_hbm.at[0, 0], u0)
    pltpu.sync_copy(u_hbm.at[1, 0], u1)
    base0 = wid * ELPT

    def _gathers(c, s):
        pltpu.async_copy(x_hbm.at[idx.at[c, 0]], r0.at[s], gsem.at[s])
        pltpu.async_copy(x_hbm.at[idx.at[c, 1]], r1.at[s], gsem.at[s])

    def _wait_gathers(c, s):
        pltpu.make_async_copy(x_hbm.at[idx.at[c, 0]], r0.at[s],
                              gsem.at[s]).wait()
        pltpu.make_async_copy(x_hbm.at[idx.at[c, 1]], r1.at[s],
                              gsem.at[s]).wait()

    def _writes(c, s):
        base = pl.multiple_of(base0 + c * KD, 8)
        pltpu.async_copy(r0.at[s], xcat_hbm.at[pl.ds(base, KD), pl.ds(0, D)],
                         wsem.at[s])
        pltpu.async_copy(r1.at[s], xcat_hbm.at[pl.ds(base, KD), pl.ds(D, D)],
                         wsem.at[s])
        pltpu.async_copy(zb.at[s], z_hbm.at[pl.ds(base, KD)], wsem.at[s])

    def _wait_writes(c, s):
        base = pl.multiple_of(base0 + c * KD, 8)
        pltpu.make_async_copy(r0.at[s], xcat_hbm.at[pl.ds(base, KD),
                                                    pl.ds(0, D)],
                              wsem.at[s]).wait()
        pltpu.make_async_copy(r1.at[s], xcat_hbm.at[pl.ds(base, KD),
                                                    pl.ds(D, D)],
                              wsem.at[s]).wait()
        pltpu.make_async_copy(zb.at[s], z_hbm.at[pl.ds(base, KD)],
                              wsem.at[s]).wait()

    for b in range(MD - 2):
        _gathers(b, b)

    def _group(g, carry):
        c0 = g * MD
        for b in range(MD):
            c = c0 + b

            @pl.when(c < nct)
            def _():
                _wait_gathers(c, b)
                for j in range(KD // LANES):
                    a0 = idx[c, 0, pl.ds(j * LANES, LANES)]
                    a1 = idx[c, 1, pl.ds(j * LANES, LANES)] - N_NODE
                    z16 = (plsc.load_gather(u0, [a0])
                           + plsc.load_gather(u1, [a1]))
                    zb[b, pl.ds(j * LANES, LANES)] = z16
                _writes(c, b)
            sg = (b + MD - 2) % MD

            @pl.when(jnp.logical_and(c >= 2, c < nct))
            def _():
                _wait_writes(c - 2, sg)

            @pl.when(c + MD - 2 < nct)
            def _():
                _gathers(c + MD - 2, sg)
        return carry

    lax.fori_loop(0, NCD // MD, _group, 0)

    @pl.when(wid == NC * NS - 1)
    def _():
        _wait_writes(_NCD_LAST - 2, (_NCD_LAST - 2) % MD)
        _wait_writes(_NCD_LAST - 1, (_NCD_LAST - 1) % MD)

    @pl.when(wid != NC * NS - 1)
    def _():
        _wait_writes(NCD - 2, (NCD - 2) % MD)
        _wait_writes(NCD - 1, (NCD - 1) % MD)


def kernel(x_chem, x_protein, edge_index_c2p, edge_index_p2c, edge_label_index,
           Wl1_c2p, bl1_c2p, Wr1_c2p, Wl1_p2c, bl1_p2c, Wr1_p2c,
           Wl2_c2p, bl2_c2p, Wr2_c2p, Wl2_p2c, bl2_p2c, Wr2_p2c,
           Wl3_c2p, bl3_c2p, Wr3_c2p, Wl3_p2c, bl3_p2c, Wr3_p2c,
           Wd, bd):
    f32 = jnp.float32
    i32 = jnp.int32
    x = jnp.concatenate([x_chem, x_protein], axis=0)
    e_cp = edge_index_c2p.astype(i32)
    e_pc = edge_index_p2c.astype(i32)
    src = jnp.concatenate([e_cp[0], e_pc[0] + N_NODE])
    dst = jnp.concatenate([e_cp[1], e_pc[1]])
    pair = jnp.stack([src.reshape(2, NS, NCA, KA),
                      dst.reshape(2, NS, NCA, KA)], axis=3)
    zero_stripe = jnp.zeros((640, D), f32)

    wd_vec = Wd.reshape(2, 1, D).astype(f32)
    bdv = jnp.concatenate([bd.astype(f32), jnp.zeros((1,), f32)])

    layers = [
        (Wl1_c2p, bl1_c2p, Wr1_c2p, Wl1_p2c, bl1_p2c, Wr1_p2c),
        (Wl2_c2p, bl2_c2p, Wr2_c2p, Wl2_p2c, bl2_p2c, Wr2_p2c),
        (Wl3_c2p, bl3_c2p, Wr3_c2p, Wl3_p2c, bl3_p2c, Wr3_p2c),
    ]
    u = None
    cnt4 = _sc_cnt(dst).reshape(2, 10, NS, 1000)
    for wl_cp, blv_cp, wr_cp, wl_pc, blv_pc, wr_pc in layers:
        wlt = jnp.stack([wl_cp, wl_pc])
        wrt = jnp.stack([wr_cp, wr_pc])
        blv = jnp.stack([blv_cp, blv_pc]).reshape(2, 1, D)
        agg = _sc_agg(x, pair, zero_stripe)
        x, u = _tc_layer(x, agg, cnt4, wlt, wrt, blv, wd_vec, bdv)

    eli = edge_label_index.astype(i32)
    pad = ELP - E_LBL
    e0p = jnp.pad(eli[0], (0, pad))
    e1p = jnp.pad(eli[1], (0, pad)) + N_NODE
    eidx = jnp.stack([e0p.reshape(NC * NS, NCD, KD),
                      e1p.reshape(NC * NS, NCD, KD)], axis=2)
    xcat, z = _sc_dec(x, eidx, u.reshape(2, 1, N_NODE))
    return (z, xcat, edge_label_index)


# speedup vs baseline: 1.0731x; 1.0731x over previous
import functools

import jax
import jax.numpy as jnp
from jax import lax
from jax.experimental import pallas as pl
from jax.experimental.pallas import tpu as pltpu
from jax.experimental.pallas import tpu_sc as plsc

NC, NS, LANES = 2, 16, 16
N_NODE = 10000
NX = 2 * N_NODE
E = 320000
E_LBL = 100000
ELP = 102400
D = 128

KA = 50
EPT = E // NS
NCA = EPT // KA
MA = 5
MI = 10

KD = 80
ELPT = ELP // (NC * NS)
NCD = ELPT // KD
MD = 4

_S15 = 15 * 624

_sc_mesh = plsc.VectorSubcoreMesh(core_axis_name="c", subcore_axis_name="s")
_sc_params = pltpu.CompilerParams(needs_layout_passes=False)


@functools.partial(
    pl.kernel,
    out_type=jax.ShapeDtypeStruct((2 * 10 * NS * 1000,), jnp.float32),
    mesh=_sc_mesh,
    scratch_types=[
        pltpu.VMEM((EPT,), jnp.int32),
        pltpu.VMEM((N_NODE,), jnp.float32),
    ],
    compiler_params=_sc_params,
)
def _sc_cnt(dst_hbm, cnt_hbm, didx, hist):
    cid = lax.axis_index("c")
    sid = lax.axis_index("s")
    base = pl.multiple_of(cid * E + sid * EPT, 8)
    pltpu.sync_copy(dst_hbm.at[pl.ds(base, EPT)], didx)
    zeros16 = jnp.zeros((LANES,), jnp.float32)
    ones16 = jnp.ones((LANES,), jnp.float32)

    def _zero(i, carry):
        hist[pl.ds(i * LANES, LANES)] = zeros16
        return carry

    lax.fori_loop(0, N_NODE // LANES, _zero, 0)

    def _accum(i, carry):
        iv = didx[pl.ds(i * LANES, LANES)]
        plsc.addupdate_scatter(hist, [iv], ones16)
        return carry

    lax.fori_loop(0, EPT // LANES, _accum, 0)
    for a in range(10):
        coff = pl.multiple_of(((cid * 10 + a) * NS + sid) * 1000, 8)
        pltpu.sync_copy(hist.at[pl.ds(a * 1000, 1000)],
                        cnt_hbm.at[pl.ds(coff, 1000)])


@functools.partial(
    pl.kernel,
    out_type=jax.ShapeDtypeStruct((2, N_NODE, D), jnp.float32),
    mesh=_sc_mesh,
    scratch_types=[
        pltpu.VMEM_SHARED((N_NODE, D), jnp.float32),
        pltpu.VMEM((MI, 2, KA), jnp.int32),
        pltpu.VMEM((MA, KA, D), jnp.float32),
        pltpu.SemaphoreType.DMA((MI,)),
        pltpu.SemaphoreType.DMA((MA,)),
        pltpu.SemaphoreType.DMA((MA,)),
    ],
    compiler_params=_sc_params,
)
def _sc_agg(x_hbm, pair_hbm, zero_hbm, agg_hbm, acc, idx, rows,
            isem, gsem, ssem):
    cid = lax.axis_index("c")
    sid = lax.axis_index("s")
    off = pl.multiple_of(sid * 624, 8)

    @pl.when(sid < 15)
    def _():
        pltpu.sync_copy(zero_hbm.at[pl.ds(0, 624)], acc.at[pl.ds(off, 624)])

    @pl.when(sid == 15)
    def _():
        pltpu.sync_copy(zero_hbm, acc.at[pl.ds(_S15, 640)])

    plsc.subcore_barrier()

    def _idx_load(c, s):
        pltpu.async_copy(pair_hbm.at[cid, sid, c], idx.at[s], isem.at[s])

    def _wait_idx(c, s):
        pltpu.make_async_copy(pair_hbm.at[cid, sid, c], idx.at[s],
                              isem.at[s]).wait()

    def _gather(c, si, sr):
        pltpu.async_copy(x_hbm.at[idx.at[si, 0]], rows.at[sr], gsem.at[sr])

    def _wait_gather(c, si, sr):
        pltpu.make_async_copy(x_hbm.at[idx.at[si, 0]], rows.at[sr],
                              gsem.at[sr]).wait()

    def _scatter(c, si, sr):
        pltpu.async_copy(rows.at[sr], acc.at[idx.at[si, 1]], ssem.at[sr],
                         add=True)

    def _wait_scatter(c, si, sr):
        pltpu.make_async_copy(rows.at[sr], acc.at[idx.at[si, 1]],
                              ssem.at[sr]).wait()

    for s in range(MI - 2):
        _idx_load(s, s)
    for b in range(MA - 2):
        _wait_idx(b, b)
        _gather(b, b, b)

    def _group(g, carry):
        c0 = g * MI
        for b in range(MI):
            c = c0 + b
            sr = b % MA
            _wait_gather(c, b, sr)
            _scatter(c, b, sr)
            srg = (b + MA - 2) % MA
            sii = (b + MI - 2) % MI

            @pl.when(c >= 2)
            def _():
                _wait_scatter(c - 2, sii, srg)

            @pl.when(c + MI - 2 < NCA)
            def _():
                _idx_load(c + MI - 2, sii)

            @pl.when(c + MA - 2 < NCA)
            def _():
                _wait_idx(c + MA - 2, (b + MA - 2) % MI)
                _gather(c + MA - 2, (b + MA - 2) % MI, srg)
        return carry

    lax.fori_loop(0, NCA // MI, _group, 0)
    _wait_scatter(NCA - 2, (NCA - 2) % MI, (NCA - 2) % MA)
    _wait_scatter(NCA - 1, (NCA - 1) % MI, (NCA - 1) % MA)
    plsc.subcore_barrier()

    @pl.when(sid < 15)
    def _():
        pltpu.sync_copy(acc.at[pl.ds(off, 624)],
                        agg_hbm.at[cid, pl.ds(off, 624)])

    @pl.when(sid == 15)
    def _():
        pltpu.sync_copy(acc.at[pl.ds(_S15, 640)],
                        agg_hbm.at[cid, pl.ds(_S15, 640)])


_BM = 2000


def _tc_layer_body(x_ref, agg_ref, cnt_ref, wlt_ref, wrt_ref, bl_ref,
                   wd_ref, bd_ref, xo_ref, u_ref):
    agg = agg_ref[0]
    cnt = jnp.concatenate(
        [jnp.sum(cnt_ref[0, a], axis=0) for a in range(_BM // 1000)])
    inv = 1.0 / jnp.maximum(cnt, 1.0)
    mean = agg * inv[:, None]
    dn = (((1,), (1,)), ((), ()))
    out = (lax.dot_general(mean, wlt_ref[0], dn,
                           preferred_element_type=jnp.float32)
           + bl_ref[0]
           + lax.dot_general(x_ref[...], wrt_ref[0], dn,
                             preferred_element_type=jnp.float32))
    out = jnp.maximum(out, 0.0)
    xo_ref[...] = out
    u = jnp.sum(out * wd_ref[0], axis=1) + bd_ref[pl.program_id(0)]
    u_ref[0, 0, :] = u


def _tc_layer(x, agg, cnt, wlt, wrt, bl, wd, bdv):
    bm = _BM
    nb = N_NODE // bm
    return pl.pallas_call(
        _tc_layer_body,
        grid=(2, nb),
        in_specs=[
            pl.BlockSpec((bm, D), lambda i, j: (i * (N_NODE // _BM) + j, 0)),
            pl.BlockSpec((1, bm, D), lambda i, j: (1 - i, j, 0)),
            pl.BlockSpec((1, _BM // 1000, NS, 1000), lambda i, j: (1 - i, j, 0, 0)),
            pl.BlockSpec((1, D, D), lambda i, j: (1 - i, 0, 0)),
            pl.BlockSpec((1, D, D), lambda i, j: (1 - i, 0, 0)),
            pl.BlockSpec((1, 1, D), lambda i, j: (1 - i, 0, 0)),
            pl.BlockSpec((1, 1, D), lambda i, j: (i, 0, 0)),
            pl.BlockSpec(memory_space=pltpu.SMEM),
        ],
        out_specs=[
            pl.BlockSpec((bm, D), lambda i, j: (i * (N_NODE // _BM) + j, 0)),
            pl.BlockSpec((1, 1, _BM), lambda i, j: (i * (N_NODE // _BM) + j, 0, 0)),
        ],
        out_shape=[
            jax.ShapeDtypeStruct((NX, D), jnp.float32),
            jax.ShapeDtypeStruct((NX // _BM, 1, _BM), jnp.float32),
        ],
    )(x, agg, cnt, wlt, wrt, bl, wd, bdv)


_NCD_LAST = (E_LBL - 31 * ELPT) // KD


@functools.partial(
    pl.kernel,
    out_type=(
        jax.ShapeDtypeStruct((E_LBL, 2 * D), jnp.float32),
        jax.ShapeDtypeStruct((E_LBL,), jnp.float32),
    ),
    mesh=_sc_mesh,
    scratch_types=[
        pltpu.VMEM((NCD, 2, KD), jnp.int32),
        pltpu.VMEM((MD, KD, D), jnp.float32),
        pltpu.VMEM((MD, KD, D), jnp.float32),
        pltpu.VMEM((N_NODE,), jnp.float32),
        pltpu.VMEM((N_NODE,), jnp.float32),
        pltpu.VMEM((MD, KD), jnp.float32),
        pltpu.SemaphoreType.DMA((MD,)),
        pltpu.SemaphoreType.DMA((MD,)),
    ],
    compiler_params=_sc_params,
)
def _sc_dec(x_hbm, eidx_hbm, u_hbm, xcat_hbm, z_hbm,
            idx, r0, r1, u0, u1, zb, gsem, wsem):
    cid = lax.axis_index("c")
    sid = lax.axis_index("s")
    wid = sid * NC + cid
    nct = jnp.where(wid == NC * NS - 1, _NCD_LAST, NCD)
    pltpu.sync_copy(eidx_hbm.at[wid], idx)
    pltpu.sync_copy(u_hbm.at[0, 0], u0)
    pltpu.sync_copy(u_hbm.at[1, 0], u1)
    base0 = wid * ELPT

    def _gathers(c, s):
        pltpu.async_copy(x_hbm.at[idx.at[c, 0]], r0.at[s], gsem.at[s])
        pltpu.async_copy(x_hbm.at[idx.at[c, 1]], r1.at[s], gsem.at[s])

    def _wait_gathers(c, s):
        pltpu.make_async_copy(x_hbm.at[idx.at[c, 0]], r0.at[s],
                              gsem.at[s]).wait()
        pltpu.make_async_copy(x_hbm.at[idx.at[c, 1]], r1.at[s],
                              gsem.at[s]).wait()

    def _writes(c, s):
        base = pl.multiple_of(base0 + c * KD, 8)
        pltpu.async_copy(r0.at[s], xcat_hbm.at[pl.ds(base, KD), pl.ds(0, D)],
                         wsem.at[s])
        pltpu.async_copy(r1.at[s], xcat_hbm.at[pl.ds(base, KD), pl.ds(D, D)],
                         wsem.at[s])
        pltpu.async_copy(zb.at[s], z_hbm.at[pl.ds(base, KD)], wsem.at[s])

    def _wait_writes(c, s):
        base = pl.multiple_of(base0 + c * KD, 8)
        pltpu.make_async_copy(r0.at[s], xcat_hbm.at[pl.ds(base, KD),
                                                    pl.ds(0, D)],
                              wsem.at[s]).wait()
        pltpu.make_async_copy(r1.at[s], xcat_hbm.at[pl.ds(base, KD),
                                                    pl.ds(D, D)],
                              wsem.at[s]).wait()
        pltpu.make_async_copy(zb.at[s], z_hbm.at[pl.ds(base, KD)],
                              wsem.at[s]).wait()

    for b in range(MD - 2):
        _gathers(b, b)

    def _group(g, carry):
        c0 = g * MD
        for b in range(MD):
            c = c0 + b

            @pl.when(c < nct)
            def _():
                _wait_gathers(c, b)
                for j in range(KD // LANES):
                    a0 = idx[c, 0, pl.ds(j * LANES, LANES)]
                    a1 = idx[c, 1, pl.ds(j * LANES, LANES)] - N_NODE
                    z16 = (plsc.load_gather(u0, [a0])
                           + plsc.load_gather(u1, [a1]))
                    zb[b, pl.ds(j * LANES, LANES)] = z16
                _writes(c, b)
            sg = (b + MD - 2) % MD

            @pl.when(jnp.logical_and(c >= 2, c < nct))
            def _():
                _wait_writes(c - 2, sg)

            @pl.when(c + MD - 2 < nct)
            def _():
                _gathers(c + MD - 2, sg)
        return carry

    lax.fori_loop(0, NCD // MD, _group, 0)

    @pl.when(wid == NC * NS - 1)
    def _():
        _wait_writes(_NCD_LAST - 2, (_NCD_LAST - 2) % MD)
        _wait_writes(_NCD_LAST - 1, (_NCD_LAST - 1) % MD)

    @pl.when(wid != NC * NS - 1)
    def _():
        _wait_writes(NCD - 2, (NCD - 2) % MD)
        _wait_writes(NCD - 1, (NCD - 1) % MD)


def kernel(x_chem, x_protein, edge_index_c2p, edge_index_p2c, edge_label_index,
           Wl1_c2p, bl1_c2p, Wr1_c2p, Wl1_p2c, bl1_p2c, Wr1_p2c,
           Wl2_c2p, bl2_c2p, Wr2_c2p, Wl2_p2c, bl2_p2c, Wr2_p2c,
           Wl3_c2p, bl3_c2p, Wr3_c2p, Wl3_p2c, bl3_p2c, Wr3_p2c,
           Wd, bd):
    f32 = jnp.float32
    i32 = jnp.int32
    x = jnp.concatenate([x_chem, x_protein], axis=0)
    e_cp = edge_index_c2p.astype(i32)
    e_pc = edge_index_p2c.astype(i32)
    src = jnp.concatenate([e_cp[0], e_pc[0] + N_NODE])
    dst = jnp.concatenate([e_cp[1], e_pc[1]])
    pair = jnp.stack([src.reshape(2, NS, NCA, KA),
                      dst.reshape(2, NS, NCA, KA)], axis=3)
    zero_stripe = jnp.zeros((640, D), f32)

    wd_vec = Wd.reshape(2, 1, D).astype(f32)
    bdv = jnp.concatenate([bd.astype(f32), jnp.zeros((1,), f32)])

    layers = [
        (Wl1_c2p, bl1_c2p, Wr1_c2p, Wl1_p2c, bl1_p2c, Wr1_p2c),
        (Wl2_c2p, bl2_c2p, Wr2_c2p, Wl2_p2c, bl2_p2c, Wr2_p2c),
        (Wl3_c2p, bl3_c2p, Wr3_c2p, Wl3_p2c, bl3_p2c, Wr3_p2c),
    ]
    u = None
    cnt4 = _sc_cnt(dst).reshape(2, 10, NS, 1000)
    for wl_cp, blv_cp, wr_cp, wl_pc, blv_pc, wr_pc in layers:
        wlt = jnp.stack([wl_cp, wl_pc])
        wrt = jnp.stack([wr_cp, wr_pc])
        blv = jnp.stack([blv_cp, blv_pc]).reshape(2, 1, D)
        agg = _sc_agg(x, pair, zero_stripe)
        x, u = _tc_layer(x, agg, cnt4, wlt, wrt, blv, wd_vec, bdv)

    eli = edge_label_index.astype(i32)
    pad = ELP - E_LBL
    e0p = jnp.pad(eli[0], (0, pad))
    e1p = jnp.pad(eli[1], (0, pad)) + N_NODE
    eidx = jnp.stack([e0p.reshape(NC * NS, NCD, KD),
                      e1p.reshape(NC * NS, NCD, KD)], axis=2)
    xcat, z = _sc_dec(x, eidx, u.reshape(2, 1, N_NODE))
    return (z, xcat, edge_label_index)


# speedup vs baseline: 1.0792x; 1.0057x over previous
import functools

import jax
import jax.numpy as jnp
from jax import lax
from jax.experimental import pallas as pl
from jax.experimental.pallas import tpu as pltpu
from jax.experimental.pallas import tpu_sc as plsc

NC, NS, LANES = 2, 16, 16
N_NODE = 10000
NX = 2 * N_NODE
E = 320000
E_LBL = 100000
ELP = 102400
D = 128

KA = 50
EPT = E // NS
NCA = EPT // KA
MA = 5
MI = 10

KD = 80
ELPT = ELP // (NC * NS)
NCD = ELPT // KD
MD = 4

_S15 = 15 * 624

_sc_mesh = plsc.VectorSubcoreMesh(core_axis_name="c", subcore_axis_name="s")
_sc_params = pltpu.CompilerParams(needs_layout_passes=False)


@functools.partial(
    pl.kernel,
    out_type=jax.ShapeDtypeStruct((2 * 10 * NS * 1000,), jnp.float32),
    mesh=_sc_mesh,
    scratch_types=[
        pltpu.VMEM((EPT,), jnp.int32),
        pltpu.VMEM((N_NODE,), jnp.float32),
    ],
    compiler_params=_sc_params,
)
def _sc_cnt(dst_hbm, cnt_hbm, didx, hist):
    cid = lax.axis_index("c")
    sid = lax.axis_index("s")
    base = pl.multiple_of(cid * E + sid * EPT, 8)
    pltpu.sync_copy(dst_hbm.at[pl.ds(base, EPT)], didx)
    zeros16 = jnp.zeros((LANES,), jnp.float32)
    ones16 = jnp.ones((LANES,), jnp.float32)

    def _zero(i, carry):
        hist[pl.ds(i * LANES, LANES)] = zeros16
        return carry

    lax.fori_loop(0, N_NODE // LANES, _zero, 0)

    def _accum(i, carry):
        iv = didx[pl.ds(i * LANES, LANES)]
        plsc.addupdate_scatter(hist, [iv], ones16)
        return carry

    lax.fori_loop(0, EPT // LANES, _accum, 0)
    for a in range(10):
        coff = pl.multiple_of(((cid * 10 + a) * NS + sid) * 1000, 8)
        pltpu.sync_copy(hist.at[pl.ds(a * 1000, 1000)],
                        cnt_hbm.at[pl.ds(coff, 1000)])


@functools.partial(
    pl.kernel,
    out_type=jax.ShapeDtypeStruct((2, N_NODE, D), jnp.float32),
    mesh=_sc_mesh,
    scratch_types=[
        pltpu.VMEM_SHARED((N_NODE, D), jnp.float32),
        pltpu.VMEM((MI, 2, KA), jnp.int32),
        pltpu.VMEM((MA, KA, D), jnp.float32),
        pltpu.SemaphoreType.DMA((MI,)),
        pltpu.SemaphoreType.DMA((MA,)),
        pltpu.SemaphoreType.DMA((MA,)),
    ],
    compiler_params=_sc_params,
)
def _sc_agg(x_hbm, pair_hbm, zero_hbm, agg_hbm, acc, idx, rows,
            isem, gsem, ssem):
    cid = lax.axis_index("c")
    sid = lax.axis_index("s")
    off = pl.multiple_of(sid * 624, 8)

    @pl.when(sid < 15)
    def _():
        pltpu.sync_copy(zero_hbm.at[pl.ds(0, 624)], acc.at[pl.ds(off, 624)])

    @pl.when(sid == 15)
    def _():
        pltpu.sync_copy(zero_hbm, acc.at[pl.ds(_S15, 640)])

    plsc.subcore_barrier()

    def _idx_load(c, s):
        pltpu.async_copy(pair_hbm.at[cid, sid, c], idx.at[s], isem.at[s])

    def _wait_idx(c, s):
        pltpu.make_async_copy(pair_hbm.at[cid, sid, c], idx.at[s],
                              isem.at[s]).wait()

    def _gather(c, si, sr):
        pltpu.async_copy(x_hbm.at[idx.at[si, 0]], rows.at[sr], gsem.at[sr])

    def _wait_gather(c, si, sr):
        pltpu.make_async_copy(x_hbm.at[idx.at[si, 0]], rows.at[sr],
                              gsem.at[sr]).wait()

    def _scatter(c, si, sr):
        pltpu.async_copy(rows.at[sr], acc.at[idx.at[si, 1]], ssem.at[sr],
                         add=True)

    def _wait_scatter(c, si, sr):
        pltpu.make_async_copy(rows.at[sr], acc.at[idx.at[si, 1]],
                              ssem.at[sr]).wait()

    for s in range(MI - 2):
        _idx_load(s, s)
    for b in range(MA - 2):
        _wait_idx(b, b)
        _gather(b, b, b)

    def _group(g, carry):
        c0 = g * MI
        for b in range(MI):
            c = c0 + b
            sr = b % MA
            _wait_gather(c, b, sr)
            _scatter(c, b, sr)
            srg = (b + MA - 2) % MA
            sii = (b + MI - 2) % MI

            @pl.when(c >= 2)
            def _():
                _wait_scatter(c - 2, sii, srg)

            @pl.when(c + MI - 2 < NCA)
            def _():
                _idx_load(c + MI - 2, sii)

            @pl.when(c + MA - 2 < NCA)
            def _():
                _wait_idx(c + MA - 2, (b + MA - 2) % MI)
                _gather(c + MA - 2, (b + MA - 2) % MI, srg)
        return carry

    lax.fori_loop(0, NCA // MI, _group, 0)
    _wait_scatter(NCA - 2, (NCA - 2) % MI, (NCA - 2) % MA)
    _wait_scatter(NCA - 1, (NCA - 1) % MI, (NCA - 1) % MA)
    plsc.subcore_barrier()

    @pl.when(sid < 15)
    def _():
        pltpu.sync_copy(acc.at[pl.ds(off, 624)],
                        agg_hbm.at[cid, pl.ds(off, 624)])

    @pl.when(sid == 15)
    def _():
        pltpu.sync_copy(acc.at[pl.ds(_S15, 640)],
                        agg_hbm.at[cid, pl.ds(_S15, 640)])


_BM = 10000


def _tc_layer_body(x_ref, agg_ref, cnt_ref, wlt_ref, wrt_ref, bl_ref,
                   wd_ref, bd_ref, xo_ref, u_ref):
    agg = agg_ref[0]
    cnt = jnp.concatenate(
        [jnp.sum(cnt_ref[0, a], axis=0) for a in range(_BM // 1000)])
    inv = 1.0 / jnp.maximum(cnt, 1.0)
    mean = agg * inv[:, None]
    dn = (((1,), (1,)), ((), ()))
    out = (lax.dot_general(mean, wlt_ref[0], dn,
                           preferred_element_type=jnp.float32)
           + bl_ref[0]
           + lax.dot_general(x_ref[...], wrt_ref[0], dn,
                             preferred_element_type=jnp.float32))
    out = jnp.maximum(out, 0.0)
    xo_ref[...] = out
    u = jnp.sum(out * wd_ref[0], axis=1) + bd_ref[pl.program_id(0)]
    u_ref[0, 0, :] = u


def _tc_layer(x, agg, cnt, wlt, wrt, bl, wd, bdv):
    bm = _BM
    nb = N_NODE // bm
    return pl.pallas_call(
        _tc_layer_body,
        grid=(2, nb),
        in_specs=[
            pl.BlockSpec((bm, D), lambda i, j: (i * (N_NODE // _BM) + j, 0)),
            pl.BlockSpec((1, bm, D), lambda i, j: (1 - i, j, 0)),
            pl.BlockSpec((1, _BM // 1000, NS, 1000), lambda i, j: (1 - i, j, 0, 0)),
            pl.BlockSpec((1, D, D), lambda i, j: (1 - i, 0, 0)),
            pl.BlockSpec((1, D, D), lambda i, j: (1 - i, 0, 0)),
            pl.BlockSpec((1, 1, D), lambda i, j: (1 - i, 0, 0)),
            pl.BlockSpec((1, 1, D), lambda i, j: (i, 0, 0)),
            pl.BlockSpec(memory_space=pltpu.SMEM),
        ],
        out_specs=[
            pl.BlockSpec((bm, D), lambda i, j: (i * (N_NODE // _BM) + j, 0)),
            pl.BlockSpec((1, 1, _BM), lambda i, j: (i * (N_NODE // _BM) + j, 0, 0)),
        ],
        out_shape=[
            jax.ShapeDtypeStruct((NX, D), jnp.float32),
            jax.ShapeDtypeStruct((NX // _BM, 1, _BM), jnp.float32),
        ],
    )(x, agg, cnt, wlt, wrt, bl, wd, bdv)


_NCD_LAST = (E_LBL - 31 * ELPT) // KD


@functools.partial(
    pl.kernel,
    out_type=(
        jax.ShapeDtypeStruct((E_LBL, 2 * D), jnp.float32),
        jax.ShapeDtypeStruct((E_LBL,), jnp.float32),
    ),
    mesh=_sc_mesh,
    scratch_types=[
        pltpu.VMEM((NCD, 2, KD), jnp.int32),
        pltpu.VMEM((MD, KD, D), jnp.float32),
        pltpu.VMEM((MD, KD, D), jnp.float32),
        pltpu.VMEM((N_NODE,), jnp.float32),
        pltpu.VMEM((N_NODE,), jnp.float32),
        pltpu.VMEM((MD, KD), jnp.float32),
        pltpu.SemaphoreType.DMA((MD,)),
        pltpu.SemaphoreType.DMA((MD,)),
    ],
    compiler_params=_sc_params,
)
def _sc_dec(x_hbm, eidx_hbm, u_hbm, xcat_hbm, z_hbm,
            idx, r0, r1, u0, u1, zb, gsem, wsem):
    cid = lax.axis_index("c")
    sid = lax.axis_index("s")
    wid = sid * NC + cid
    nct = jnp.where(wid == NC * NS - 1, _NCD_LAST, NCD)
    pltpu.sync_copy(eidx_hbm.at[wid], idx)
    pltpu.sync_copy(u_hbm.at[0, 0], u0)
    pltpu.sync_copy(u_hbm.at[1, 0], u1)
    base0 = wid * ELPT

    def _gathers(c, s):
        pltpu.async_copy(x_hbm.at[idx.at[c, 0]], r0.at[s], gsem.at[s])
        pltpu.async_copy(x_hbm.at[idx.at[c, 1]], r1.at[s], gsem.at[s])

    def _wait_gathers(c, s):
        pltpu.make_async_copy(x_hbm.at[idx.at[c, 0]], r0.at[s],
                              gsem.at[s]).wait()
        pltpu.make_async_copy(x_hbm.at[idx.at[c, 1]], r1.at[s],
                              gsem.at[s]).wait()

    def _writes(c, s):
        base = pl.multiple_of(base0 + c * KD, 8)
        pltpu.async_copy(r0.at[s], xcat_hbm.at[pl.ds(base, KD), pl.ds(0, D)],
                         wsem.at[s])
        pltpu.async_copy(r1.at[s], xcat_hbm.at[pl.ds(base, KD), pl.ds(D, D)],
                         wsem.at[s])
        pltpu.async_copy(zb.at[s], z_hbm.at[pl.ds(base, KD)], wsem.at[s])

    def _wait_writes(c, s):
        base = pl.multiple_of(base0 + c * KD, 8)
        pltpu.make_async_copy(r0.at[s], xcat_hbm.at[pl.ds(base, KD),
                                                    pl.ds(0, D)],
                              wsem.at[s]).wait()
        pltpu.make_async_copy(r1.at[s], xcat_hbm.at[pl.ds(base, KD),
                                                    pl.ds(D, D)],
                              wsem.at[s]).wait()
        pltpu.make_async_copy(zb.at[s], z_hbm.at[pl.ds(base, KD)],
                              wsem.at[s]).wait()

    for b in range(MD - 2):
        _gathers(b, b)

    def _group(g, carry):
        c0 = g * MD
        for b in range(MD):
            c = c0 + b

            @pl.when(c < nct)
            def _():
                _wait_gathers(c, b)
                for j in range(KD // LANES):
                    a0 = idx[c, 0, pl.ds(j * LANES, LANES)]
                    a1 = idx[c, 1, pl.ds(j * LANES, LANES)] - N_NODE
                    z16 = (plsc.load_gather(u0, [a0])
                           + plsc.load_gather(u1, [a1]))
                    zb[b, pl.ds(j * LANES, LANES)] = z16
                _writes(c, b)
            sg = (b + MD - 2) % MD

            @pl.when(jnp.logical_and(c >= 2, c < nct))
            def _():
                _wait_writes(c - 2, sg)

            @pl.when(c + MD - 2 < nct)
            def _():
                _gathers(c + MD - 2, sg)
        return carry

    lax.fori_loop(0, NCD // MD, _group, 0)

    @pl.when(wid == NC * NS - 1)
    def _():
        _wait_writes(_NCD_LAST - 2, (_NCD_LAST - 2) % MD)
        _wait_writes(_NCD_LAST - 1, (_NCD_LAST - 1) % MD)

    @pl.when(wid != NC * NS - 1)
    def _():
        _wait_writes(NCD - 2, (NCD - 2) % MD)
        _wait_writes(NCD - 1, (NCD - 1) % MD)


def kernel(x_chem, x_protein, edge_index_c2p, edge_index_p2c, edge_label_index,
           Wl1_c2p, bl1_c2p, Wr1_c2p, Wl1_p2c, bl1_p2c, Wr1_p2c,
           Wl2_c2p, bl2_c2p, Wr2_c2p, Wl2_p2c, bl2_p2c, Wr2_p2c,
           Wl3_c2p, bl3_c2p, Wr3_c2p, Wl3_p2c, bl3_p2c, Wr3_p2c,
           Wd, bd):
    f32 = jnp.float32
    i32 = jnp.int32
    x = jnp.concatenate([x_chem, x_protein], axis=0)
    e_cp = edge_index_c2p.astype(i32)
    e_pc = edge_index_p2c.astype(i32)
    src = jnp.concatenate([e_cp[0], e_pc[0] + N_NODE])
    dst = jnp.concatenate([e_cp[1], e_pc[1]])
    pair = jnp.stack([src.reshape(2, NS, NCA, KA),
                      dst.reshape(2, NS, NCA, KA)], axis=3)
    zero_stripe = jnp.zeros((640, D), f32)

    wd_vec = Wd.reshape(2, 1, D).astype(f32)
    bdv = jnp.concatenate([bd.astype(f32), jnp.zeros((1,), f32)])

    layers = [
        (Wl1_c2p, bl1_c2p, Wr1_c2p, Wl1_p2c, bl1_p2c, Wr1_p2c),
        (Wl2_c2p, bl2_c2p, Wr2_c2p, Wl2_p2c, bl2_p2c, Wr2_p2c),
        (Wl3_c2p, bl3_c2p, Wr3_c2p, Wl3_p2c, bl3_p2c, Wr3_p2c),
    ]
    u = None
    cnt4 = _sc_cnt(dst).reshape(2, 10, NS, 1000)
    for wl_cp, blv_cp, wr_cp, wl_pc, blv_pc, wr_pc in layers:
        wlt = jnp.stack([wl_cp, wl_pc])
        wrt = jnp.stack([wr_cp, wr_pc])
        blv = jnp.stack([blv_cp, blv_pc]).reshape(2, 1, D)
        agg = _sc_agg(x, pair, zero_stripe)
        x, u = _tc_layer(x, agg, cnt4, wlt, wrt, blv, wd_vec, bdv)

    eli = edge_label_index.astype(i32)
    pad = ELP - E_LBL
    e0p = jnp.pad(eli[0], (0, pad))
    e1p = jnp.pad(eli[1], (0, pad)) + N_NODE
    eidx = jnp.stack([e0p.reshape(NC * NS, NCD, KD),
                      e1p.reshape(NC * NS, NCD, KD)], axis=2)
    xcat, z = _sc_dec(x, eidx, u.reshape(2, 1, N_NODE))
    return (z, xcat, edge_label_index)
